# baseline (speedup 1.0000x reference)
import functools

import jax
import jax.numpy as jnp
from jax import lax
from jax.experimental import pallas as pl
from jax.experimental.pallas import tpu as pltpu
from jax.experimental.pallas import tpu_sc as plsc

_NC = 2
_NS = 16
_L = 16
_PD = 128


def _sc_pool(x_flat, tablep, batch, seq, dim):
    nw = _NC * _NS
    rows_per_w = batch // nw
    cb = 8
    steps = rows_per_w // cb
    inv_seq = 1.0 / seq
    nd = dim // _L
    unroll = 10
    chunk = 80
    nchunk = (cb * seq) // chunk
    assert seq % unroll == 0 and steps % 2 == 0 and cb * seq % chunk == 0
    assert chunk % 8 == 0 and chunk <= 128
    mesh = plsc.VectorSubcoreMesh(core_axis_name="c", subcore_axis_name="s")

    @functools.partial(
        pl.kernel,
        mesh=mesh,
        compiler_params=pltpu.CompilerParams(use_tc_tiling_on_sc=True),
        out_type=jax.ShapeDtypeStruct((batch, _PD), jnp.float32),
        scratch_types=[
            pltpu.VMEM((nchunk, chunk), jnp.int32),
            pltpu.VMEM((nchunk, chunk), jnp.int32),
            pltpu.VMEM((cb * seq, _PD), jnp.float32),
            pltpu.VMEM((cb * seq, _PD), jnp.float32),
            pltpu.VMEM((cb, _PD), jnp.float32),
            pltpu.VMEM((cb, _PD), jnp.float32),
            pltpu.SemaphoreType.DMA,
            pltpu.SemaphoreType.DMA,
            pltpu.SemaphoreType.DMA,
            pltpu.SemaphoreType.DMA,
            pltpu.SemaphoreType.DMA,
            pltpu.SemaphoreType.DMA,
        ],
    )
    def pool_kernel(x_hbm, table_hbm, out_hbm, idx0, idx1, rows0, rows1,
                    oacc0, oacc1, sg0, sg1, si0, si1, so0, so1):
        wid = lax.axis_index("s") * _NC + lax.axis_index("c")
        base = wid * rows_per_w

        def load_idx_sync(idx_v, gg):
            off = (base + gg * cb) * seq
            for k in range(nchunk):
                pltpu.sync_copy(
                    x_hbm.at[pl.ds(off + k * chunk, chunk)], idx_v.at[k])

        def load_idx_async(idx_v, gg, si):
            off = (base + gg * cb) * seq
            for k in range(nchunk):
                pltpu.async_copy(
                    x_hbm.at[pl.ds(off + k * chunk, chunk)], idx_v.at[k], si)

        def wait_idx(idx_v, si):
            for k in range(nchunk):
                pltpu.make_async_copy(
                    x_hbm.at[pl.ds(0, chunk)], idx_v.at[k], si).wait()

        def start_gathers(idx_v, rows_v, sg):
            for k in range(nchunk):
                pltpu.async_copy(
                    table_hbm.at[idx_v.at[k]],
                    rows_v.at[pl.ds(k * chunk, chunk)], sg)

        def wait_gathers(rows_v, sg):
            pltpu.make_async_copy(
                table_hbm.at[pl.ds(0, cb * seq)], rows_v, sg).wait()

        for slot, idx_v, rows_v, sg in ((0, idx0, rows0, sg0),
                                        (1, idx1, rows1, sg1)):
            load_idx_sync(idx_v, slot)
            start_gathers(idx_v, rows_v, sg)

        @pl.loop(0, steps, step=2)
        def _(g):
            for slot, idx_v, rows_v, oacc, sg, si, so in (
                    (0, idx0, rows0, oacc0, sg0, si0, so0),
                    (1, idx1, rows1, oacc1, sg1, si1, so1)):
                gg = g + slot
                wait_gathers(rows_v, sg)
                @pl.when(gg + 2 < steps)
                def _():
                    load_idx_async(idx_v, gg + 2, si)
                @pl.when(gg >= 2)
                def _():
                    pltpu.make_async_copy(
                        oacc, out_hbm.at[pl.ds(0, cb)], so).wait()
                for c in range(cb):
                    def body(s, accs, c=c):
                        for j in range(unroll):
                            r = c * seq + s * unroll + j
                            accs = tuple(
                                accs[d] + rows_v[r, pl.ds(d * _L, _L)]
                                for d in range(nd))
                        return accs
                    accs = lax.fori_loop(
                        0, seq // unroll, body,
                        tuple(jnp.zeros((_L,), jnp.float32)
                              for _ in range(nd)))
                    for d in range(nd):
                        oacc[c, pl.ds(d * _L, _L)] = accs[d] * inv_seq
                pltpu.async_copy(
                    oacc, out_hbm.at[pl.ds(base + gg * cb, cb)], so)

                @pl.when(gg + 2 < steps)
                def _():
                    wait_idx(idx_v, si)
                    start_gathers(idx_v, rows_v, sg)

        pltpu.make_async_copy(oacc0, out_hbm.at[pl.ds(0, cb)], so0).wait()
        pltpu.make_async_copy(oacc1, out_hbm.at[pl.ds(0, cb)], so1).wait()

    return pool_kernel(x_flat, tablep)


def _transpose_pad(table_t, vocab, dim):
    nw = _NC * _NS
    w_cols = 256
    nchunks = vocab // w_cols
    tail = vocab - nchunks * w_cols
    tail0 = nchunks * w_cols
    max_steps = (nchunks + nw - 1) // nw
    mesh = plsc.VectorSubcoreMesh(core_axis_name="c", subcore_axis_name="s")

    @functools.partial(
        pl.kernel,
        mesh=mesh,
        compiler_params=pltpu.CompilerParams(use_tc_tiling_on_sc=True,
                                             needs_layout_passes=False),
        out_type=jax.ShapeDtypeStruct((vocab, _PD), jnp.float32),
        scratch_types=[
            pltpu.VMEM((dim, w_cols), jnp.float32),
            pltpu.VMEM((dim, w_cols), jnp.float32),
            pltpu.VMEM((w_cols, _PD), jnp.float32),
            pltpu.VMEM((w_cols, _PD), jnp.float32),
            pltpu.VMEM((dim, max(tail, 1)), jnp.float32),
            pltpu.SemaphoreType.DMA,
            pltpu.SemaphoreType.DMA,
            pltpu.SemaphoreType.DMA,
            pltpu.SemaphoreType.DMA,
        ],
    )
    def tr_kernel(t_hbm, t2_hbm, o_hbm, in0, in1, ob0, ob1, in_t,
                  si0, si1, so0, so1):
        wid = lax.axis_index("s") * _NC + lax.axis_index("c")
        lane = lax.iota(jnp.int32, 16)

        def cid_of(step):
            return wid + step * nw

        def start_in(in_v, step, si):
            @pl.when(cid_of(step) < nchunks)
            def _():
                pltpu.async_copy(
                    t_hbm.at[:, pl.ds(cid_of(step) * w_cols, w_cols)],
                    in_v, si)

        def wait_in(in_v, si):
            pltpu.make_async_copy(
                t_hbm.at[:, pl.ds(0, w_cols)], in_v, si).wait()

        start_in(in0, 0, si0)
        start_in(in1, 1, si1)

        @pl.loop(0, max_steps, step=2)
        def _(g):
            for slot, in_v, ob, si, so in ((0, in0, ob0, si0, so0),
                                           (1, in1, ob1, si1, so1)):
                step = g + slot
                cid = cid_of(step)

                @pl.when(cid < nchunks)
                def _():
                    wait_in(in_v, si)
                    @pl.when(step >= 2)
                    def _():
                        pltpu.make_async_copy(
                            ob, o_hbm.at[pl.ds(0, w_cols)], so).wait()

                    @pl.loop(0, w_cols)
                    def _(v):
                        for d0 in range(0, dim, _L):
                            val = plsc.load_gather(
                                in_v, [d0 + lane,
                                       jnp.broadcast_to(v, (16,))])
                            ob[v, pl.ds(d0, _L)] = val
                    pltpu.async_copy(
                        ob, o_hbm.at[pl.ds(cid * w_cols, w_cols)], so)
                    start_in(in_v, step + 2, si)

        for ob, so, slot in ((ob0, so0, 0), (ob1, so1, 1)):
            @pl.when(cid_of(slot) < nchunks)
            def _():
                pltpu.make_async_copy(
                    ob, o_hbm.at[pl.ds(0, w_cols)], so).wait()

        if tail:
            @pl.when(wid == 0)
            def _():
                pltpu.sync_copy(t2_hbm, in_t)

                @pl.loop(0, tail)
                def _(v):
                    for d0 in range(0, dim, _L):
                        val = plsc.load_gather(
                            in_t, [d0 + lane, jnp.broadcast_to(v, (16,))])
                        ob0[v, pl.ds(d0, _L)] = val
                pltpu.sync_copy(ob0.at[pl.ds(0, tail)],
                                o_hbm.at[pl.ds(tail0, tail)])

    return tr_kernel(table_t, lax.slice(table_t, (0, tail0), (dim, vocab)))


def _tc_head(pooled, w, b2d):
    batch = pooled.shape[0]
    seq, dim = w.shape
    bb = 2048

    def head_kernel(p_ref, w_ref, b_ref, o_ref):
        logits = lax.dot_general(
            p_ref[...][:, :dim], w_ref[...],
            (((1,), (1,)), ((), ())),
            preferred_element_type=jnp.float32,
        ) + b_ref[...]
        o_ref[...] = 1.0 / (1.0 + jnp.exp(-logits))

    return pl.pallas_call(
        head_kernel,
        grid=(batch // bb,),
        in_specs=[
            pl.BlockSpec((bb, _PD), lambda i: (i, 0)),
            pl.BlockSpec((seq, dim), lambda i: (0, 0)),
            pl.BlockSpec((1, seq), lambda i: (0, 0)),
        ],
        out_specs=pl.BlockSpec((bb, seq), lambda i: (i, 0)),
        out_shape=jax.ShapeDtypeStruct((batch, seq), jnp.float32),
    )(pooled, w, b2d)


@jax.jit
def kernel(x, table, W, b):
    batch, seq = x.shape
    vocab, dim = table.shape
    tablep = _transpose_pad(table.T, vocab, dim)
    pooled = _sc_pool(x.reshape(batch * seq), tablep, batch, seq, dim)
    return _tc_head(pooled, W, b.reshape(1, seq))


# speedup vs baseline: 3.0553x; 3.0553x over previous
import functools

import jax
import jax.numpy as jnp
from jax import lax
from jax.experimental import pallas as pl
from jax.experimental.pallas import tpu as pltpu
from jax.experimental.pallas import tpu_sc as plsc

_NC = 2
_NS = 16
_L = 16
_PD = 128


def _sc_pool(x_flat, tablep, batch, seq, dim):
    nw = _NC * _NS
    rows_per_w = batch // nw
    cb = 8
    steps = rows_per_w // cb
    inv_seq = 1.0 / seq
    nd = dim // _L
    unroll = 10
    chunk = 80
    nchunk = (cb * seq) // chunk
    assert seq % unroll == 0 and steps % 2 == 0 and cb * seq % chunk == 0
    assert chunk % 8 == 0 and chunk <= 128
    mesh = plsc.VectorSubcoreMesh(core_axis_name="c", subcore_axis_name="s")

    @functools.partial(
        pl.kernel,
        mesh=mesh,
        compiler_params=pltpu.CompilerParams(use_tc_tiling_on_sc=True),
        out_type=jax.ShapeDtypeStruct((batch, _PD), jnp.float32),
        scratch_types=[
            pltpu.VMEM((nchunk, chunk), jnp.int32),
            pltpu.VMEM((nchunk, chunk), jnp.int32),
            pltpu.VMEM((cb * seq, _PD), jnp.float32),
            pltpu.VMEM((cb * seq, _PD), jnp.float32),
            pltpu.VMEM((cb, _PD), jnp.float32),
            pltpu.VMEM((cb, _PD), jnp.float32),
            pltpu.SemaphoreType.DMA,
            pltpu.SemaphoreType.DMA,
            pltpu.SemaphoreType.DMA,
            pltpu.SemaphoreType.DMA,
            pltpu.SemaphoreType.DMA,
            pltpu.SemaphoreType.DMA,
        ],
    )
    def pool_kernel(x_hbm, table_hbm, out_hbm, idx0, idx1, rows0, rows1,
                    oacc0, oacc1, sg0, sg1, si0, si1, so0, so1):
        wid = lax.axis_index("s") * _NC + lax.axis_index("c")
        base = wid * rows_per_w

        def load_idx_sync(idx_v, gg):
            off = (base + gg * cb) * seq
            for k in range(nchunk):
                pltpu.sync_copy(
                    x_hbm.at[pl.ds(off + k * chunk, chunk)], idx_v.at[k])

        def load_idx_async(idx_v, gg, si):
            off = (base + gg * cb) * seq
            for k in range(nchunk):
                pltpu.async_copy(
                    x_hbm.at[pl.ds(off + k * chunk, chunk)], idx_v.at[k], si)

        def wait_idx(idx_v, si):
            for k in range(nchunk):
                pltpu.make_async_copy(
                    x_hbm.at[pl.ds(0, chunk)], idx_v.at[k], si).wait()

        def start_gathers(idx_v, rows_v, sg):
            for k in range(nchunk):
                pltpu.async_copy(
                    table_hbm.at[idx_v.at[k]],
                    rows_v.at[pl.ds(k * chunk, chunk)], sg)

        def wait_gathers(rows_v, sg):
            pltpu.make_async_copy(
                table_hbm.at[pl.ds(0, cb * seq)], rows_v, sg).wait()

        for slot, idx_v, rows_v, sg in ((0, idx0, rows0, sg0),
                                        (1, idx1, rows1, sg1)):
            load_idx_sync(idx_v, slot)
            start_gathers(idx_v, rows_v, sg)

        @pl.loop(0, steps, step=2)
        def _(g):
            for slot, idx_v, rows_v, oacc, sg, si, so in (
                    (0, idx0, rows0, oacc0, sg0, si0, so0),
                    (1, idx1, rows1, oacc1, sg1, si1, so1)):
                gg = g + slot
                wait_gathers(rows_v, sg)
                @pl.when(gg + 2 < steps)
                def _():
                    load_idx_async(idx_v, gg + 2, si)
                @pl.when(gg >= 2)
                def _():
                    pltpu.make_async_copy(
                        oacc, out_hbm.at[pl.ds(0, cb)], so).wait()
                for c in range(cb):
                    def body(s, accs, c=c):
                        for j in range(unroll):
                            r = c * seq + s * unroll + j
                            accs = tuple(
                                accs[d] + rows_v[r, pl.ds(d * _L, _L)]
                                for d in range(nd))
                        return accs
                    accs = lax.fori_loop(
                        0, seq // unroll, body,
                        tuple(jnp.zeros((_L,), jnp.float32)
                              for _ in range(nd)))
                    for d in range(nd):
                        oacc[c, pl.ds(d * _L, _L)] = accs[d] * inv_seq
                pltpu.async_copy(
                    oacc, out_hbm.at[pl.ds(base + gg * cb, cb)], so)

                @pl.when(gg + 2 < steps)
                def _():
                    wait_idx(idx_v, si)
                    start_gathers(idx_v, rows_v, sg)

        pltpu.make_async_copy(oacc0, out_hbm.at[pl.ds(0, cb)], so0).wait()
        pltpu.make_async_copy(oacc1, out_hbm.at[pl.ds(0, cb)], so1).wait()

    return pool_kernel(x_flat, tablep)


def _transpose_pad(table_t, vocab, dim):
    bv = 4096
    grid = (vocab + bv - 1) // bv

    def tr_kernel(t_ref, o_ref):
        o_ref[:, :dim] = t_ref[...].T

    return pl.pallas_call(
        tr_kernel,
        grid=(grid,),
        in_specs=[pl.BlockSpec((dim, bv), lambda i: (0, i))],
        out_specs=pl.BlockSpec((bv, _PD), lambda i: (i, 0)),
        out_shape=jax.ShapeDtypeStruct((vocab, _PD), jnp.float32),
    )(table_t)


def _tc_head(pooled, w, b2d):
    batch = pooled.shape[0]
    seq, dim = w.shape
    bb = 2048

    def head_kernel(p_ref, w_ref, b_ref, o_ref):
        logits = lax.dot_general(
            p_ref[...][:, :dim], w_ref[...],
            (((1,), (1,)), ((), ())),
            preferred_element_type=jnp.float32,
        ) + b_ref[...]
        o_ref[...] = 1.0 / (1.0 + jnp.exp(-logits))

    return pl.pallas_call(
        head_kernel,
        grid=(batch // bb,),
        in_specs=[
            pl.BlockSpec((bb, _PD), lambda i: (i, 0)),
            pl.BlockSpec((seq, dim), lambda i: (0, 0)),
            pl.BlockSpec((1, seq), lambda i: (0, 0)),
        ],
        out_specs=pl.BlockSpec((bb, seq), lambda i: (i, 0)),
        out_shape=jax.ShapeDtypeStruct((batch, seq), jnp.float32),
    )(pooled, w, b2d)


@jax.jit
def kernel(x, table, W, b):
    batch, seq = x.shape
    vocab, dim = table.shape
    tablep = _transpose_pad(table.T, vocab, dim)
    pooled = _sc_pool(x.reshape(batch * seq), tablep, batch, seq, dim)
    return _tc_head(pooled, W, b.reshape(1, seq))


# speedup vs baseline: 3.5132x; 1.1499x over previous
import functools

import jax
import jax.numpy as jnp
from jax import lax
from jax.experimental import pallas as pl
from jax.experimental.pallas import tpu as pltpu
from jax.experimental.pallas import tpu_sc as plsc

_NC = 2
_NS = 16
_L = 16
_PD = 128


def _sc_pool(x_flat, tablep, batch, seq, dim):
    nw = _NC * _NS
    rows_per_w = batch // nw
    cb = 8
    steps = rows_per_w // cb
    inv_seq = 1.0 / seq
    nd = dim // _L
    unroll = 10
    chunk = 80
    nchunk = (cb * seq) // chunk
    assert seq % unroll == 0 and steps % 2 == 0 and cb * seq % chunk == 0
    assert chunk % 8 == 0 and chunk <= 128
    mesh = plsc.VectorSubcoreMesh(core_axis_name="c", subcore_axis_name="s")

    @functools.partial(
        pl.kernel,
        mesh=mesh,
        compiler_params=pltpu.CompilerParams(use_tc_tiling_on_sc=True),
        out_type=jax.ShapeDtypeStruct((batch, _PD), jnp.float32),
        scratch_types=[
            pltpu.VMEM((nchunk, chunk), jnp.int32),
            pltpu.VMEM((nchunk, chunk), jnp.int32),
            pltpu.VMEM((cb * seq, _PD), jnp.float32),
            pltpu.VMEM((cb * seq, _PD), jnp.float32),
            pltpu.VMEM((cb, _PD), jnp.float32),
            pltpu.VMEM((cb, _PD), jnp.float32),
            pltpu.SemaphoreType.DMA,
            pltpu.SemaphoreType.DMA,
            pltpu.SemaphoreType.DMA,
            pltpu.SemaphoreType.DMA,
            pltpu.SemaphoreType.DMA,
            pltpu.SemaphoreType.DMA,
        ],
    )
    def pool_kernel(x_hbm, table_hbm, out_hbm, idx0, idx1, rows0, rows1,
                    oacc0, oacc1, sg0, sg1, si0, si1, so0, so1):
        wid = lax.axis_index("s") * _NC + lax.axis_index("c")
        base = wid * rows_per_w

        def load_idx_sync(idx_v, gg):
            off = (base + gg * cb) * seq
            for k in range(nchunk):
                pltpu.sync_copy(
                    x_hbm.at[pl.ds(off + k * chunk, chunk)], idx_v.at[k])

        def load_idx_async(idx_v, gg, si):
            off = (base + gg * cb) * seq
            for k in range(nchunk):
                pltpu.async_copy(
                    x_hbm.at[pl.ds(off + k * chunk, chunk)], idx_v.at[k], si)

        def wait_idx(idx_v, si):
            for k in range(nchunk):
                pltpu.make_async_copy(
                    x_hbm.at[pl.ds(0, chunk)], idx_v.at[k], si).wait()

        def start_gathers(idx_v, rows_v, sg):
            for k in range(nchunk):
                pltpu.async_copy(
                    table_hbm.at[idx_v.at[k]],
                    rows_v.at[pl.ds(k * chunk, chunk)], sg)

        def wait_gathers(rows_v, sg):
            pltpu.make_async_copy(
                table_hbm.at[pl.ds(0, cb * seq)], rows_v, sg).wait()

        for slot, idx_v, rows_v, sg in ((0, idx0, rows0, sg0),
                                        (1, idx1, rows1, sg1)):
            load_idx_sync(idx_v, slot)
            start_gathers(idx_v, rows_v, sg)

        @pl.loop(0, steps, step=2)
        def _(g):
            for slot, idx_v, rows_v, oacc, sg, si, so in (
                    (0, idx0, rows0, oacc0, sg0, si0, so0),
                    (1, idx1, rows1, oacc1, sg1, si1, so1)):
                gg = g + slot
                wait_gathers(rows_v, sg)
                @pl.when(gg + 2 < steps)
                def _():
                    load_idx_async(idx_v, gg + 2, si)
                @pl.when(gg >= 2)
                def _():
                    pltpu.make_async_copy(
                        oacc, out_hbm.at[pl.ds(0, cb)], so).wait()
                for c in range(cb):
                    def body(s, accs, c=c):
                        for j in range(unroll):
                            r = c * seq + s * unroll + j
                            accs = tuple(
                                accs[d] + rows_v[r, pl.ds(d * _L, _L)]
                                for d in range(nd))
                        return accs
                    accs = lax.fori_loop(
                        0, seq // unroll, body,
                        tuple(jnp.zeros((_L,), jnp.float32)
                              for _ in range(nd)))
                    for d in range(nd):
                        oacc[c, pl.ds(d * _L, _L)] = accs[d] * inv_seq
                pltpu.async_copy(
                    oacc, out_hbm.at[pl.ds(base + gg * cb, cb)], so)

                @pl.when(gg + 2 < steps)
                def _():
                    wait_idx(idx_v, si)
                    start_gathers(idx_v, rows_v, sg)

        pltpu.make_async_copy(oacc0, out_hbm.at[pl.ds(0, cb)], so0).wait()
        pltpu.make_async_copy(oacc1, out_hbm.at[pl.ds(0, cb)], so1).wait()

    return pool_kernel(x_flat, tablep)


def _transpose_pad(table_t, vocab, dim):
    bv = 8192
    grid = (vocab + bv - 1) // bv

    def tr_kernel(t_ref, o_ref):
        o_ref[:, :dim] = t_ref[...].T

    return pl.pallas_call(
        tr_kernel,
        grid=(grid,),
        in_specs=[pl.BlockSpec((dim, bv), lambda i: (0, i))],
        out_specs=pl.BlockSpec((bv, _PD), lambda i: (i, 0)),
        out_shape=jax.ShapeDtypeStruct((vocab, _PD), jnp.float32),
    )(table_t)


def _tc_head(pooled, w, b2d):
    batch = pooled.shape[0]
    seq, dim = w.shape
    bb = 2048

    def head_kernel(p_ref, w_ref, b_ref, o_ref):
        logits = lax.dot_general(
            p_ref[...][:, :dim], w_ref[...],
            (((1,), (1,)), ((), ())),
            preferred_element_type=jnp.float32,
        ) + b_ref[...]
        o_ref[...] = 1.0 / (1.0 + jnp.exp(-logits))

    return pl.pallas_call(
        head_kernel,
        grid=(batch // bb,),
        in_specs=[
            pl.BlockSpec((bb, _PD), lambda i: (i, 0)),
            pl.BlockSpec((seq, dim), lambda i: (0, 0)),
            pl.BlockSpec((1, seq), lambda i: (0, 0)),
        ],
        out_specs=pl.BlockSpec((bb, seq), lambda i: (i, 0)),
        out_shape=jax.ShapeDtypeStruct((batch, seq), jnp.float32),
    )(pooled, w, b2d)


@jax.jit
def kernel(x, table, W, b):
    batch, seq = x.shape
    vocab, dim = table.shape
    tablep = _transpose_pad(table.T, vocab, dim)
    pooled = _sc_pool(x.reshape(batch * seq), tablep, batch, seq, dim)
    return _tc_head(pooled, W, b.reshape(1, seq))


# speedup vs baseline: 3.6850x; 1.0489x over previous
import functools

import jax
import jax.numpy as jnp
from jax import lax
from jax.experimental import pallas as pl
from jax.experimental.pallas import tpu as pltpu
from jax.experimental.pallas import tpu_sc as plsc

_NC = 2
_NS = 16
_L = 16
_PD = 128


def _sc_pool(x_flat, tablep, batch, seq, dim):
    nw = _NC * _NS
    rows_per_w = batch // nw
    cb = 8
    steps = rows_per_w // cb
    inv_seq = 1.0 / seq
    nd = dim // _L
    unroll = 10
    chunk = 80
    nchunk = (cb * seq) // chunk
    assert seq % unroll == 0 and steps % 2 == 0 and cb * seq % chunk == 0
    assert chunk % 8 == 0 and chunk <= 128
    mesh = plsc.VectorSubcoreMesh(core_axis_name="c", subcore_axis_name="s")

    @functools.partial(
        pl.kernel,
        mesh=mesh,
        compiler_params=pltpu.CompilerParams(use_tc_tiling_on_sc=True),
        out_type=jax.ShapeDtypeStruct((batch, _PD), jnp.float32),
        scratch_types=[
            pltpu.VMEM((nchunk, chunk), jnp.int32),
            pltpu.VMEM((nchunk, chunk), jnp.int32),
            pltpu.VMEM((cb * seq, _PD), jnp.float32),
            pltpu.VMEM((cb * seq, _PD), jnp.float32),
            pltpu.VMEM((cb, _PD), jnp.float32),
            pltpu.VMEM((cb, _PD), jnp.float32),
            pltpu.SemaphoreType.DMA,
            pltpu.SemaphoreType.DMA,
            pltpu.SemaphoreType.DMA,
            pltpu.SemaphoreType.DMA,
            pltpu.SemaphoreType.DMA,
            pltpu.SemaphoreType.DMA,
        ],
    )
    def pool_kernel(x_hbm, table_hbm, out_hbm, idx0, idx1, rows0, rows1,
                    oacc0, oacc1, sg0, sg1, si0, si1, so0, so1):
        wid = lax.axis_index("s") * _NC + lax.axis_index("c")
        base = wid * rows_per_w

        def load_idx_sync(idx_v, gg):
            off = (base + gg * cb) * seq
            for k in range(nchunk):
                pltpu.sync_copy(
                    x_hbm.at[pl.ds(off + k * chunk, chunk)], idx_v.at[k])

        def load_idx_async(idx_v, gg, si):
            off = (base + gg * cb) * seq
            for k in range(nchunk):
                pltpu.async_copy(
                    x_hbm.at[pl.ds(off + k * chunk, chunk)], idx_v.at[k], si)

        def wait_idx(idx_v, si):
            for k in range(nchunk):
                pltpu.make_async_copy(
                    x_hbm.at[pl.ds(0, chunk)], idx_v.at[k], si).wait()

        def start_gathers(idx_v, rows_v, sg):
            for k in range(nchunk):
                pltpu.async_copy(
                    table_hbm.at[idx_v.at[k]],
                    rows_v.at[pl.ds(k * chunk, chunk)], sg)

        def wait_gathers(rows_v, sg):
            pltpu.make_async_copy(
                table_hbm.at[pl.ds(0, cb * seq)], rows_v, sg).wait()

        for slot, idx_v, rows_v, sg in ((0, idx0, rows0, sg0),
                                        (1, idx1, rows1, sg1)):
            load_idx_sync(idx_v, slot)
            start_gathers(idx_v, rows_v, sg)

        @pl.loop(0, steps, step=2)
        def _(g):
            for slot, idx_v, rows_v, oacc, sg, si, so in (
                    (0, idx0, rows0, oacc0, sg0, si0, so0),
                    (1, idx1, rows1, oacc1, sg1, si1, so1)):
                gg = g + slot
                wait_gathers(rows_v, sg)
                @pl.when(gg + 2 < steps)
                def _():
                    load_idx_async(idx_v, gg + 2, si)
                @pl.when(gg >= 2)
                def _():
                    pltpu.make_async_copy(
                        oacc, out_hbm.at[pl.ds(0, cb)], so).wait()
                for c in range(cb):
                    def body(s, accs, c=c):
                        for j in range(unroll):
                            r = c * seq + s * unroll + j
                            accs = tuple(
                                accs[d] + rows_v[r, pl.ds(d * _L, _L)]
                                for d in range(nd))
                        return accs
                    accs = lax.fori_loop(
                        0, seq // unroll, body,
                        tuple(jnp.zeros((_L,), jnp.float32)
                              for _ in range(nd)))
                    for d in range(nd):
                        oacc[c, pl.ds(d * _L, _L)] = accs[d] * inv_seq
                pltpu.async_copy(
                    oacc, out_hbm.at[pl.ds(base + gg * cb, cb)], so)

                @pl.when(gg + 2 < steps)
                def _():
                    wait_idx(idx_v, si)
                    start_gathers(idx_v, rows_v, sg)

        pltpu.make_async_copy(oacc0, out_hbm.at[pl.ds(0, cb)], so0).wait()
        pltpu.make_async_copy(oacc1, out_hbm.at[pl.ds(0, cb)], so1).wait()

    return pool_kernel(x_flat, tablep)


def _transpose_pad(table_t, vocab, dim):
    bv = 16384
    grid = (vocab + bv - 1) // bv

    def tr_kernel(t_ref, o_ref):
        o_ref[:, :dim] = t_ref[...].T

    return pl.pallas_call(
        tr_kernel,
        grid=(grid,),
        in_specs=[pl.BlockSpec((dim, bv), lambda i: (0, i))],
        out_specs=pl.BlockSpec((bv, _PD), lambda i: (i, 0)),
        out_shape=jax.ShapeDtypeStruct((vocab, _PD), jnp.float32),
    )(table_t)


def _tc_head(pooled, w, b2d):
    batch = pooled.shape[0]
    seq, dim = w.shape
    bb = 2048

    def head_kernel(p_ref, w_ref, b_ref, o_ref):
        logits = lax.dot_general(
            p_ref[...][:, :dim], w_ref[...],
            (((1,), (1,)), ((), ())),
            preferred_element_type=jnp.float32,
        ) + b_ref[...]
        o_ref[...] = 1.0 / (1.0 + jnp.exp(-logits))

    return pl.pallas_call(
        head_kernel,
        grid=(batch // bb,),
        in_specs=[
            pl.BlockSpec((bb, _PD), lambda i: (i, 0)),
            pl.BlockSpec((seq, dim), lambda i: (0, 0)),
            pl.BlockSpec((1, seq), lambda i: (0, 0)),
        ],
        out_specs=pl.BlockSpec((bb, seq), lambda i: (i, 0)),
        out_shape=jax.ShapeDtypeStruct((batch, seq), jnp.float32),
    )(pooled, w, b2d)


@jax.jit
def kernel(x, table, W, b):
    batch, seq = x.shape
    vocab, dim = table.shape
    tablep = _transpose_pad(table.T, vocab, dim)
    pooled = _sc_pool(x.reshape(batch * seq), tablep, batch, seq, dim)
    return _tc_head(pooled, W, b.reshape(1, seq))
